# baseline (speedup 1.0000x reference)
import functools

import jax
import jax.numpy as jnp
from jax import lax
from jax.experimental import pallas as pl
from jax.experimental.pallas import tpu as pltpu
from jax.experimental.pallas import tpu_sc as plsc

B = 4
V = 100000
F = 200000
W = 16
NC = 2
NS = 16
NW = NC * NS
K = 128
CHUNKS = 50
FT = K * CHUNKS
F_PAD = NW * FT
F_ALLOC = F_PAD + 2 * K
V_PAD = 100352
RPS = V_PAD // NS
PK = 64
PIECES = RPS // PK
GPC = K // 16

_mesh = plsc.VectorSubcoreMesh(core_axis_name="c", subcore_axis_name="s")


def _sc_scratch():
    s = [pltpu.VMEM_SHARED((V_PAD, W), jnp.float32)]
    s += [pltpu.VMEM((PK, 3), jnp.float32) for _ in range(2 * B)]
    s += [pltpu.VMEM((PK, W), jnp.float32) for _ in range(2)]
    s += [pltpu.VMEM((K, W), jnp.float32) for _ in range(6)]
    s += [pltpu.VMEM((K, W), jnp.float32) for _ in range(2)]
    s += [pltpu.VMEM((K, 3), jnp.int32) for _ in range(2)]
    s += [pltpu.VMEM((K,), jnp.int32) for _ in range(18)]
    s += [pltpu.SemaphoreType.DMA for _ in range(6)]
    return s


@functools.partial(
    pl.kernel,
    out_type=(jax.ShapeDtypeStruct((NC, V_PAD, W), jnp.float32),
              jax.ShapeDtypeStruct((NC * V_PAD, W), jnp.float32)),
    mesh=_mesh,
    scratch_types=_sc_scratch(),
    compiler_params=pltpu.CompilerParams(use_tc_tiling_on_sc=False,
                                         needs_layout_passes=False),
)
def _face_scatter(vvpad, fpad, out_hbm, vpack, *rest):
    acc = rest[0]
    pb = rest[1:9]
    ob = rest[9:11]
    gb = rest[11:17]
    nb = rest[17:19]
    fb = rest[19:21]
    ix = rest[21:27]
    gix = rest[27:33]
    sidx = rest[33:39]
    sem_g0, sem_g1, sem_s0, sem_s1, sem_f0, sem_f1 = rest[39:45]
    sem_g = (sem_g0, sem_g1)
    sem_s = (sem_s0, sem_s1)
    sem_f = (sem_f0, sem_f1)

    cid = lax.axis_index("c")
    sid = lax.axis_index("s")
    tid = cid * NS + sid
    row0 = sid * RPS
    vbase = cid * V_PAD
    lanes = lax.iota(jnp.int32, 16)

    def _znb(k, carry):
        nb[0][k] = jnp.zeros((W,), jnp.float32)
        nb[1][k] = jnp.zeros((W,), jnp.float32)
        return carry
    lax.fori_loop(0, K, _znb, 0)

    def _zacc(i, carry):
        pltpu.sync_copy(nb[0], acc.at[pl.ds(row0 + i * K, K)])
        return carry
    lax.fori_loop(0, RPS // K, _zacc, 0)

    def _fire_pack_in(slot, p):
        pp = jnp.minimum(p, PIECES - 1)
        for b in range(B):
            pltpu.async_copy(vvpad.at[b, pl.ds(row0 + pp * PK, PK)],
                             pb[slot * B + b], sem_g[slot])

    def _wait_pack_in(slot):
        for b in range(B):
            pltpu.make_async_copy(vvpad.at[b, pl.ds(0, PK)],
                                  pb[slot * B + b], sem_g[slot]).wait()

    for slot in range(2):
        for gg in range(PK // 16):
            rows = gg * 16 + lanes
            for col in range(B * 3, W):
                plsc.store_scatter(ob[slot],
                                   [rows, jnp.full((16,), col, jnp.int32)],
                                   jnp.zeros((16,), jnp.float32))

    _fire_pack_in(0, 0)

    def _pack_pair(t, carry):
        for s in range(2):
            p = 2 * t + s
            _fire_pack_in(1 - s, p + 1)
            _wait_pack_in(s)

            @pl.when(t >= 1)
            def _():
                pltpu.make_async_copy(
                    ob[s], vpack.at[pl.ds(vbase, PK)], sem_s[s]).wait()

            for gg in range(PK // 16):
                rows = gg * 16 + lanes
                for b in range(B):
                    for cc in range(3):
                        vals = plsc.load_gather(
                            pb[s * B + b],
                            [rows, jnp.full((16,), cc, jnp.int32)])
                        plsc.store_scatter(
                            ob[s],
                            [rows, jnp.full((16,), b * 3 + cc, jnp.int32)],
                            vals)
            pltpu.async_copy(
                ob[s], vpack.at[pl.ds(vbase + row0 + p * PK, PK)], sem_s[s])
        return carry
    lax.fori_loop(0, PIECES // 2, _pack_pair, 0)

    for s in range(2):
        pltpu.make_async_copy(ob[s], vpack.at[pl.ds(vbase, PK)],
                              sem_s[s]).wait()
    _wait_pack_in(0)

    plsc.subcore_barrier()

    fbase = tid * FT

    def _fire_fb(slot, j):
        pltpu.async_copy(fpad.at[pl.ds(fbase + j * K, K)], fb[slot],
                         sem_f[slot])

    def _wait_fb(slot):
        pltpu.make_async_copy(fpad.at[pl.ds(0, K)], fb[slot],
                              sem_f[slot]).wait()

    def _extract(slot):
        for gg in range(GPC):
            rows = gg * 16 + lanes
            for c in range(3):
                vals = plsc.load_gather(
                    fb[slot], [rows, jnp.full((16,), c, jnp.int32)])
                ix[slot * 3 + c][pl.ds(gg * 16, 16)] = vals
                gix[slot * 3 + c][pl.ds(gg * 16, 16)] = vals + vbase

    def _fire_gathers(slot):
        for c in range(3):
            pltpu.async_copy(vpack.at[gix[slot * 3 + c]], gb[slot * 3 + c],
                             sem_g[slot])

    def _wait_gathers(slot):
        for c in range(3):
            pltpu.make_async_copy(vpack.at[gix[slot * 3 + c]],
                                  gb[slot * 3 + c], sem_g[slot]).wait()

    def _fire_scatters(slot):
        for c in range(3):
            pltpu.async_copy(nb[slot], acc.at[sidx[slot * 3 + c]],
                             sem_s[slot], add=True)

    def _wait_scatters(slot):
        for c in range(3):
            pltpu.make_async_copy(nb[slot], acc.at[sidx[slot * 3 + c]],
                                  sem_s[slot]).wait()

    def _compute(slot):
        for gg in range(GPC):
            rows = gg * 16 + lanes
            for b in range(B):
                cols = [jnp.full((16,), b * 3 + cc, jnp.int32)
                        for cc in range(3)]
                v0 = [plsc.load_gather(gb[slot * 3 + 0], [rows, cols[cc]])
                      for cc in range(3)]
                v1 = [plsc.load_gather(gb[slot * 3 + 1], [rows, cols[cc]])
                      for cc in range(3)]
                v2 = [plsc.load_gather(gb[slot * 3 + 2], [rows, cols[cc]])
                      for cc in range(3)]
                e1 = [v0[cc] - v1[cc] for cc in range(3)]
                e2 = [v2[cc] - v1[cc] for cc in range(3)]
                for cc in range(3):
                    n = (e2[(cc + 1) % 3] * e1[(cc + 2) % 3]
                         - e2[(cc + 2) % 3] * e1[(cc + 1) % 3])
                    plsc.store_scatter(nb[slot], [rows, cols[cc]], n)

    def _copy_sidx(slot):
        for c in range(3):
            for gg in range(GPC):
                sl = pl.ds(gg * 16, 16)
                sidx[slot * 3 + c][sl] = ix[slot * 3 + c][sl]

    pltpu.sync_copy(fpad.at[pl.ds(fbase, K)], fb[0])
    _extract(0)
    _fire_gathers(0)
    _fire_fb(1, 1)

    def _pair(t, carry):
        for s in range(2):
            j = 2 * t + s
            _wait_fb(1 - s)
            _extract(1 - s)
            _fire_gathers(1 - s)
            _fire_fb(s, j + 2)
            _wait_gathers(s)

            @pl.when(j >= 2)
            def _():
                _wait_scatters(s)

            _compute(s)
            _copy_sidx(s)
            _fire_scatters(s)
        return carry
    lax.fori_loop(0, CHUNKS // 2, _pair, 0)

    _wait_gathers(0)
    _wait_fb(1)
    _wait_scatters(0)
    _wait_scatters(1)

    plsc.subcore_barrier()
    pltpu.sync_copy(acc.at[pl.ds(row0, RPS)],
                    out_hbm.at[cid, pl.ds(row0, RPS)])


_RB = 2000


def _combine_body(p0_ref, p1_ref, o_ref):
    s = p0_ref[...] + p1_ref[...]
    sq = s * s
    ii = lax.broadcasted_iota(jnp.int32, (W, W), 0)
    jj = lax.broadcasted_iota(jnp.int32, (W, W), 1)
    g = ((ii // 3 == jj // 3) & (ii < B * 3) & (jj < B * 3)).astype(jnp.float32)
    gs = jnp.dot(sq, g, preferred_element_type=jnp.float32)
    n = s * lax.rsqrt(jnp.maximum(gs, 1e-12))
    parts = [n[:, 3 * b:3 * b + 3].reshape(1, _RB, 3) for b in range(B)]
    o_ref[...] = jnp.concatenate(parts, axis=0)


_combine = pl.pallas_call(
    _combine_body,
    out_shape=jax.ShapeDtypeStruct((B, V, 3), jnp.float32),
    grid=(V // _RB,),
    in_specs=[
        pl.BlockSpec((_RB, W), lambda i: (i, 0)),
        pl.BlockSpec((_RB, W), lambda i: (i, 0)),
    ],
    out_specs=pl.BlockSpec((B, _RB, 3), lambda i: (0, i, 0)),
)


def kernel(vertices, faces):
    vvpad = jnp.zeros((B, V_PAD, 3), jnp.float32).at[:, :V].set(vertices)
    fpad = jnp.zeros((F_ALLOC, 3), jnp.int32).at[:F].set(faces)
    partials, _ = _face_scatter(vvpad, fpad)
    return _combine(partials[0], partials[1])


# speedup vs baseline: 2.1996x; 2.1996x over previous
import functools

import jax
import jax.numpy as jnp
from jax import lax
from jax.experimental import pallas as pl
from jax.experimental.pallas import tpu as pltpu
from jax.experimental.pallas import tpu_sc as plsc

B = 4
V = 100000
F = 200000
W = 16
NC = 2
NS = 16
NW = NC * NS
K = 128
CHUNKS = 100
FT = K * CHUNKS
F_PAD = NS * FT
F_ALLOC = F_PAD + 2 * K
V_PAD = 100352
RPS = V_PAD // NS
PK = 64
PIECES = RPS // PK
GPC = K // 16

_mesh = plsc.VectorSubcoreMesh(core_axis_name="c", subcore_axis_name="s")


def _sc_scratch():
    s = [pltpu.VMEM_SHARED((V_PAD, W), jnp.float32)]
    s += [pltpu.VMEM((PK,), jnp.float32) for _ in range(12)]
    s += [pltpu.VMEM((PK, W), jnp.float32) for _ in range(2)]
    s += [pltpu.VMEM((K, W), jnp.float32) for _ in range(6)]
    s += [pltpu.VMEM((K, W), jnp.float32) for _ in range(2)]
    s += [pltpu.VMEM((K,), jnp.int32) for _ in range(18)]
    s += [pltpu.VMEM((K, 3), jnp.int32) for _ in range(2)]
    s += [pltpu.VMEM((PK, W), jnp.float32) for _ in range(2)]
    s += [pltpu.VMEM((2, 3, PK), jnp.float32) for _ in range(2)]
    s += [pltpu.SemaphoreType.DMA for _ in range(6)]
    return s


@functools.partial(
    pl.kernel,
    out_type=(jax.ShapeDtypeStruct((NC, 2, 3, V_PAD), jnp.float32),
              jax.ShapeDtypeStruct((NC * V_PAD, W), jnp.float32)),
    mesh=_mesh,
    scratch_types=_sc_scratch(),
    compiler_params=pltpu.CompilerParams(use_tc_tiling_on_sc=False,
                                         needs_layout_passes=False),
)
def _vertex_normals_sc(v12, frows, o_hbm, pack, *rest):
    acc = rest[0]
    pc = rest[1:13]
    pp = rest[13:15]
    gb = rest[15:21]
    nb = rest[21:23]
    ix = rest[23:29]
    gix = rest[29:35]
    sidx = rest[35:41]
    fbst = rest[41:43]
    ab = rest[43:45]
    ob = rest[45:47]
    sem_a, sem_b, sem_s0, sem_s1, sem_f0, sem_f1 = rest[47:53]
    sem_g = (sem_a, sem_b)
    sem_s = (sem_s0, sem_s1)
    sem_f = (sem_f0, sem_f1)

    cid = lax.axis_index("c")
    sid = lax.axis_index("s")
    tid = cid * NS + sid
    row0 = sid * RPS
    fbase = sid * FT
    vbase = cid * V_PAD
    lanes = lax.iota(jnp.int32, 16)
    zrow = jnp.zeros((W,), jnp.float32)

    def _zrows(k, carry):
        nb[0][k] = zrow
        nb[1][k] = zrow
        return carry
    lax.fori_loop(0, K, _zrows, 0)

    def _zpp(k, carry):
        pp[0][k] = zrow
        pp[1][k] = zrow
        return carry
    lax.fori_loop(0, PK, _zpp, 0)

    def _zacc(i, carry):
        pltpu.sync_copy(nb[0], acc.at[pl.ds(row0 + i * K, K)])
        return carry
    lax.fori_loop(0, RPS // K, _zacc, 0)

    def _fire_pc(slot, p):
        ps = jnp.minimum(p, PIECES - 1)
        r = row0 + ps * PK
        for q in range(2):
            for c in range(3):
                pr = c * 4 + 2 * cid + q
                pltpu.async_copy(v12.at[pr, pl.ds(r, PK)],
                                 pc[slot * 6 + q * 3 + c], sem_g[slot])

    def _wait_pc(slot):
        for i in range(6):
            pltpu.make_async_copy(v12.at[0, pl.ds(0, PK)],
                                  pc[slot * 6 + i], sem_g[slot]).wait()

    _fire_pc(0, 0)

    def _pack_pair(t, carry):
        for s in range(2):
            p = 2 * t + s
            _fire_pc(1 - s, p + 1)
            _wait_pc(s)

            @pl.when(t >= 1)
            def _():
                pltpu.make_async_copy(pp[s], pack.at[pl.ds(0, PK)],
                                      sem_s[s]).wait()

            for gg in range(PK // 16):
                rows = gg * 16 + lanes
                for lp in range(6):
                    vals = pc[s * 6 + lp][pl.ds(gg * 16, 16)]
                    plsc.store_scatter(pp[s],
                                       [rows, jnp.full((16,), lp, jnp.int32)],
                                       vals)
            pltpu.async_copy(pp[s],
                             pack.at[pl.ds(vbase + row0 + p * PK, PK)],
                             sem_s[s])
        return carry
    lax.fori_loop(0, PIECES // 2, _pack_pair, 0)

    for s in range(2):
        pltpu.make_async_copy(pp[s], pack.at[pl.ds(0, PK)], sem_s[s]).wait()
    _wait_pc(0)

    plsc.subcore_barrier()

    def _fire_ix(slot, j):
        pltpu.async_copy(frows.at[pl.ds(fbase + j * K, K)], fbst[slot],
                         sem_f[slot])

    def _wait_ix(slot):
        pltpu.make_async_copy(frows.at[pl.ds(0, K)], fbst[slot],
                              sem_f[slot]).wait()

    def _mk_gix(slot):
        for gg in range(GPC):
            rows = gg * 16 + lanes
            sl = pl.ds(gg * 16, 16)
            for c in range(3):
                vals = plsc.load_gather(
                    fbst[slot], [rows, jnp.full((16,), c, jnp.int32)])
                ix[slot * 3 + c][sl] = vals
                gix[slot * 3 + c][sl] = vals + vbase

    def _fire_gathers(slot):
        for c in range(3):
            pltpu.async_copy(pack.at[gix[slot * 3 + c]], gb[slot * 3 + c],
                             sem_g[slot])

    def _wait_gathers(slot):
        for c in range(3):
            pltpu.make_async_copy(pack.at[gix[slot * 3 + c]],
                                  gb[slot * 3 + c], sem_g[slot]).wait()

    def _fire_scatters(slot):
        for c in range(3):
            pltpu.async_copy(nb[slot], acc.at[sidx[slot * 3 + c]],
                             sem_s[slot], add=True)

    def _wait_scatters(slot):
        for c in range(3):
            pltpu.make_async_copy(nb[slot], acc.at[sidx[slot * 3 + c]],
                                  sem_s[slot]).wait()

    def _compute(slot):
        for gg in range(GPC):
            rows = gg * 16 + lanes
            for q in range(2):
                cols = [jnp.full((16,), q * 3 + cc, jnp.int32)
                        for cc in range(3)]
                v0 = [plsc.load_gather(gb[slot * 3 + 0], [rows, cols[cc]])
                      for cc in range(3)]
                v1 = [plsc.load_gather(gb[slot * 3 + 1], [rows, cols[cc]])
                      for cc in range(3)]
                v2 = [plsc.load_gather(gb[slot * 3 + 2], [rows, cols[cc]])
                      for cc in range(3)]
                e1 = [v0[cc] - v1[cc] for cc in range(3)]
                e2 = [v2[cc] - v1[cc] for cc in range(3)]
                for cc in range(3):
                    n = (e2[(cc + 1) % 3] * e1[(cc + 2) % 3]
                         - e2[(cc + 2) % 3] * e1[(cc + 1) % 3])
                    plsc.store_scatter(nb[slot], [rows, cols[cc]], n)

    def _copy_sidx(slot):
        for c in range(3):
            for gg in range(GPC):
                sl = pl.ds(gg * 16, 16)
                sidx[slot * 3 + c][sl] = ix[slot * 3 + c][sl]

    _fire_ix(0, 0)
    _wait_ix(0)
    _mk_gix(0)
    _fire_gathers(0)
    _fire_ix(1, 1)

    def _pair(t, carry):
        for s in range(2):
            j = 2 * t + s
            _wait_ix(1 - s)
            _mk_gix(1 - s)
            _fire_gathers(1 - s)
            _wait_gathers(s)

            @pl.when(t >= 1)
            def _():
                _wait_scatters(s)

            _compute(s)
            _copy_sidx(s)
            _fire_ix(s, j + 2)
            _fire_scatters(s)
        return carry
    lax.fori_loop(0, CHUNKS // 2, _pair, 0)

    _wait_gathers(0)
    _wait_ix(1)
    _wait_scatters(0)
    _wait_scatters(1)

    plsc.subcore_barrier()

    def _rsqrt(x):
        xi = plsc.bitcast(x, jnp.int32)
        yi = jnp.full((16,), 0x5F3759DF, jnp.int32) - \
            lax.shift_right_logical(xi, 1)
        y = plsc.bitcast(yi, jnp.float32)
        h = x * 0.5
        for _ in range(3):
            y = y * (1.5 - h * y * y)
        return y

    def _fire_ab(slot, p):
        ps = jnp.minimum(p, PIECES - 1)
        pltpu.async_copy(acc.at[pl.ds(row0 + ps * PK, PK)], ab[slot],
                         sem_g[slot])

    def _wait_ab(slot):
        pltpu.make_async_copy(acc.at[pl.ds(0, PK)], ab[slot],
                              sem_g[slot]).wait()

    _fire_ab(0, 0)

    def _norm_pair(t, carry):
        for s in range(2):
            p = 2 * t + s
            _fire_ab(1 - s, p + 1)
            _wait_ab(s)

            @pl.when(t >= 1)
            def _():
                pltpu.make_async_copy(ob[s], o_hbm.at[0, :, :, pl.ds(0, PK)],
                                      sem_s[s]).wait()

            for gg in range(PK // 16):
                rows = gg * 16 + lanes
                for q in range(2):
                    x = plsc.load_gather(
                        ab[s], [rows, jnp.full((16,), q * 3, jnp.int32)])
                    y = plsc.load_gather(
                        ab[s], [rows, jnp.full((16,), q * 3 + 1, jnp.int32)])
                    z = plsc.load_gather(
                        ab[s], [rows, jnp.full((16,), q * 3 + 2, jnp.int32)])
                    ss = jnp.maximum(x * x + y * y + z * z,
                                     jnp.full((16,), 1e-12, jnp.float32))
                    r = _rsqrt(ss)
                    ob[s][q, 0, pl.ds(gg * 16, 16)] = x * r
                    ob[s][q, 1, pl.ds(gg * 16, 16)] = y * r
                    ob[s][q, 2, pl.ds(gg * 16, 16)] = z * r
            pltpu.async_copy(ob[s],
                             o_hbm.at[cid, :, :, pl.ds(row0 + p * PK, PK)],
                             sem_s[s])
        return carry
    lax.fori_loop(0, PIECES // 2, _norm_pair, 0)

    for s in range(2):
        pltpu.make_async_copy(ob[s], o_hbm.at[0, :, :, pl.ds(0, PK)],
                              sem_s[s]).wait()
    _wait_ab(0)


def kernel(vertices, faces):
    v12 = jnp.zeros((B * 3, V_PAD), jnp.float32).at[:, :V].set(
        jnp.transpose(vertices, (2, 0, 1)).reshape(B * 3, V))
    frows = jnp.zeros((F_ALLOC, 3), jnp.int32).at[:F].set(faces)
    o, _ = _vertex_normals_sc(v12, frows)
    o = o.reshape(B, 3, V_PAD)[:, :, :V]
    return jnp.transpose(o, (0, 2, 1))


# speedup vs baseline: 3.8132x; 1.7336x over previous
import functools

import jax
import jax.numpy as jnp
from jax import lax
from jax.experimental import pallas as pl
from jax.experimental.pallas import tpu as pltpu
from jax.experimental.pallas import tpu_sc as plsc

B = 4
V = 100000
F = 200000
W = 16
NC = 2
NS = 16
NW = NC * NS
K = 128
CHUNKS = 100
FT = K * CHUNKS
F_PAD = NS * FT
F_ALLOC = F_PAD + 2 * K
V_PAD = 100352
RPS = V_PAD // NS
PK = 64
PIECES = RPS // PK
GPC = K // 16

_mesh = plsc.VectorSubcoreMesh(core_axis_name="c", subcore_axis_name="s")


def _sc_scratch():
    s = [pltpu.VMEM_SHARED((V_PAD, W), jnp.float32)]
    s += [pltpu.VMEM((PK,), jnp.float32) for _ in range(12)]
    s += [pltpu.VMEM((PK, W), jnp.float32) for _ in range(2)]
    s += [pltpu.VMEM((K, W), jnp.float32) for _ in range(6)]
    s += [pltpu.VMEM((K, W), jnp.float32) for _ in range(2)]
    s += [pltpu.VMEM((K,), jnp.int32) for _ in range(18)]
    s += [pltpu.VMEM((PK, W), jnp.float32) for _ in range(2)]
    s += [pltpu.VMEM((2, 3, PK), jnp.float32) for _ in range(2)]
    s += [pltpu.SemaphoreType.DMA for _ in range(6)]
    return s


@functools.partial(
    pl.kernel,
    out_type=(jax.ShapeDtypeStruct((NC, 2, 3, V_PAD), jnp.float32),
              jax.ShapeDtypeStruct((NC * V_PAD, W), jnp.float32)),
    mesh=_mesh,
    scratch_types=_sc_scratch(),
    compiler_params=pltpu.CompilerParams(use_tc_tiling_on_sc=False,
                                         needs_layout_passes=False),
)
def _vertex_normals_sc(v12, f3, o_hbm, pack, *rest):
    acc = rest[0]
    pc = rest[1:13]
    pp = rest[13:15]
    gb = rest[15:21]
    nb = rest[21:23]
    ix = rest[23:29]
    gix = rest[29:35]
    sidx = rest[35:41]
    ab = rest[41:43]
    ob = rest[43:45]
    sem_a, sem_b, sem_s0, sem_s1, sem_f0, sem_f1 = rest[45:51]
    sem_g = (sem_a, sem_b)
    sem_s = (sem_s0, sem_s1)
    sem_f = (sem_f0, sem_f1)

    cid = lax.axis_index("c")
    sid = lax.axis_index("s")
    tid = cid * NS + sid
    row0 = sid * RPS
    fbase = sid * FT
    vbase = cid * V_PAD
    lanes = lax.iota(jnp.int32, 16)
    zrow = jnp.zeros((W,), jnp.float32)

    def _zrows(k, carry):
        nb[0][k] = zrow
        nb[1][k] = zrow
        return carry
    lax.fori_loop(0, K, _zrows, 0)

    def _zpp(k, carry):
        pp[0][k] = zrow
        pp[1][k] = zrow
        return carry
    lax.fori_loop(0, PK, _zpp, 0)

    def _zacc(i, carry):
        pltpu.sync_copy(nb[0], acc.at[pl.ds(row0 + i * K, K)])
        return carry
    lax.fori_loop(0, RPS // K, _zacc, 0)

    def _fire_pc(slot, p):
        ps = jnp.minimum(p, PIECES - 1)
        r = row0 + ps * PK
        for q in range(2):
            for c in range(3):
                pr = c * 4 + 2 * cid + q
                pltpu.async_copy(v12.at[pr, pl.ds(r, PK)],
                                 pc[slot * 6 + q * 3 + c], sem_g[slot])

    def _wait_pc(slot):
        for i in range(6):
            pltpu.make_async_copy(v12.at[0, pl.ds(0, PK)],
                                  pc[slot * 6 + i], sem_g[slot]).wait()

    _fire_pc(0, 0)

    def _pack_pair(t, carry):
        for s in range(2):
            p = 2 * t + s
            _fire_pc(1 - s, p + 1)
            _wait_pc(s)

            @pl.when(t >= 1)
            def _():
                pltpu.make_async_copy(pp[s], pack.at[pl.ds(0, PK)],
                                      sem_s[s]).wait()

            for gg in range(PK // 16):
                rows = gg * 16 + lanes
                for lp in range(6):
                    vals = pc[s * 6 + lp][pl.ds(gg * 16, 16)]
                    plsc.store_scatter(pp[s],
                                       [rows, jnp.full((16,), lp, jnp.int32)],
                                       vals)
            pltpu.async_copy(pp[s],
                             pack.at[pl.ds(vbase + row0 + p * PK, PK)],
                             sem_s[s])
        return carry
    lax.fori_loop(0, PIECES // 2, _pack_pair, 0)

    for s in range(2):
        pltpu.make_async_copy(pp[s], pack.at[pl.ds(0, PK)], sem_s[s]).wait()
    _wait_pc(0)

    plsc.subcore_barrier()

    def _fire_ix(slot, j):
        for c in range(3):
            pltpu.async_copy(f3.at[c, pl.ds(fbase + j * K, K)],
                             ix[slot * 3 + c], sem_f[slot])

    def _wait_ix(slot):
        for c in range(3):
            pltpu.make_async_copy(f3.at[0, pl.ds(0, K)],
                                  ix[slot * 3 + c], sem_f[slot]).wait()

    def _mk_gix(slot):
        for c in range(3):
            for gg in range(GPC):
                sl = pl.ds(gg * 16, 16)
                gix[slot * 3 + c][sl] = ix[slot * 3 + c][sl] + vbase

    def _fire_gathers(slot):
        for c in range(3):
            pltpu.async_copy(pack.at[gix[slot * 3 + c]], gb[slot * 3 + c],
                             sem_g[slot])

    def _wait_gathers(slot):
        for c in range(3):
            pltpu.make_async_copy(pack.at[gix[slot * 3 + c]],
                                  gb[slot * 3 + c], sem_g[slot]).wait()

    def _fire_scatters(slot):
        for c in range(3):
            pltpu.async_copy(nb[slot], acc.at[sidx[slot * 3 + c]],
                             sem_s[slot], add=True)

    def _wait_scatters(slot):
        for c in range(3):
            pltpu.make_async_copy(nb[slot], acc.at[sidx[slot * 3 + c]],
                                  sem_s[slot]).wait()

    def _copy_sidx(slot):
        for c in range(3):
            for gg in range(GPC):
                sl = pl.ds(gg * 16, 16)
                sidx[slot * 3 + c][sl] = ix[slot * 3 + c][sl]

    def _compute(slot):
        for gg in range(GPC):
            rows = gg * 16 + lanes
            for q in range(2):
                cols = [jnp.full((16,), q * 3 + cc, jnp.int32)
                        for cc in range(3)]
                v0 = [plsc.load_gather(gb[slot * 3 + 0], [rows, cols[cc]])
                      for cc in range(3)]
                v1 = [plsc.load_gather(gb[slot * 3 + 1], [rows, cols[cc]])
                      for cc in range(3)]
                v2 = [plsc.load_gather(gb[slot * 3 + 2], [rows, cols[cc]])
                      for cc in range(3)]
                e1 = [v0[cc] - v1[cc] for cc in range(3)]
                e2 = [v2[cc] - v1[cc] for cc in range(3)]
                for cc in range(3):
                    n = (e2[(cc + 1) % 3] * e1[(cc + 2) % 3]
                         - e2[(cc + 2) % 3] * e1[(cc + 1) % 3])
                    plsc.store_scatter(nb[slot], [rows, cols[cc]], n)

    _fire_ix(0, 0)
    _wait_ix(0)
    _mk_gix(0)
    _fire_gathers(0)
    _fire_ix(1, 1)

    def _pair(t, carry):
        for s in range(2):
            j = 2 * t + s
            _wait_ix(1 - s)
            _mk_gix(1 - s)
            _fire_gathers(1 - s)
            _wait_gathers(s)

            @pl.when(t >= 1)
            def _():
                _wait_scatters(s)

            _compute(s)
            _copy_sidx(s)
            _fire_ix(s, j + 2)
            _fire_scatters(s)
        return carry
    lax.fori_loop(0, CHUNKS // 2, _pair, 0)

    _wait_gathers(0)
    _wait_ix(1)
    _wait_scatters(0)
    _wait_scatters(1)

    plsc.subcore_barrier()

    def _rsqrt(x):
        xi = plsc.bitcast(x, jnp.int32)
        yi = jnp.full((16,), 0x5F3759DF, jnp.int32) - \
            lax.shift_right_logical(xi, 1)
        y = plsc.bitcast(yi, jnp.float32)
        h = x * 0.5
        for _ in range(3):
            y = y * (1.5 - h * y * y)
        return y

    def _fire_ab(slot, p):
        ps = jnp.minimum(p, PIECES - 1)
        pltpu.async_copy(acc.at[pl.ds(row0 + ps * PK, PK)], ab[slot],
                         sem_g[slot])

    def _wait_ab(slot):
        pltpu.make_async_copy(acc.at[pl.ds(0, PK)], ab[slot],
                              sem_g[slot]).wait()

    _fire_ab(0, 0)

    def _norm_pair(t, carry):
        for s in range(2):
            p = 2 * t + s
            _fire_ab(1 - s, p + 1)
            _wait_ab(s)

            @pl.when(t >= 1)
            def _():
                pltpu.make_async_copy(ob[s], o_hbm.at[0, :, :, pl.ds(0, PK)],
                                      sem_s[s]).wait()

            for gg in range(PK // 16):
                rows = gg * 16 + lanes
                for q in range(2):
                    x = plsc.load_gather(
                        ab[s], [rows, jnp.full((16,), q * 3, jnp.int32)])
                    y = plsc.load_gather(
                        ab[s], [rows, jnp.full((16,), q * 3 + 1, jnp.int32)])
                    z = plsc.load_gather(
                        ab[s], [rows, jnp.full((16,), q * 3 + 2, jnp.int32)])
                    ss = jnp.maximum(x * x + y * y + z * z,
                                     jnp.full((16,), 1e-12, jnp.float32))
                    r = _rsqrt(ss)
                    ob[s][q, 0, pl.ds(gg * 16, 16)] = x * r
                    ob[s][q, 1, pl.ds(gg * 16, 16)] = y * r
                    ob[s][q, 2, pl.ds(gg * 16, 16)] = z * r
            pltpu.async_copy(ob[s],
                             o_hbm.at[cid, :, :, pl.ds(row0 + p * PK, PK)],
                             sem_s[s])
        return carry
    lax.fori_loop(0, PIECES // 2, _norm_pair, 0)

    for s in range(2):
        pltpu.make_async_copy(ob[s], o_hbm.at[0, :, :, pl.ds(0, PK)],
                              sem_s[s]).wait()
    _wait_ab(0)


def kernel(vertices, faces):
    v12 = jnp.zeros((B * 3, V_PAD), jnp.float32).at[:, :V].set(
        jnp.transpose(vertices, (2, 0, 1)).reshape(B * 3, V))
    f3 = jnp.zeros((3, F_ALLOC), jnp.int32).at[:, :F].set(faces.T)
    o, _ = _vertex_normals_sc(v12, f3)
    o = o.reshape(B, 3, V_PAD)[:, :, :V]
    return jnp.transpose(o, (0, 2, 1))
